# baseline (speedup 1.0000x reference)
import functools

import jax
import jax.numpy as jnp
from jax import lax
from jax.experimental import pallas as pl
from jax.experimental.pallas import tpu as pltpu
from jax.experimental.pallas import tpu_sc as plsc

N = 100000
NHID = 128
NC = 2
NS = 16
NW = NC * NS
C = 128
FULL = N // C
CPW = FULL // NW
EXTRA = FULL - CPW * NW
TAIL_ROWS = N - FULL * C
TAILW = NW - 1

_mesh = plsc.VectorSubcoreMesh(core_axis_name="c", subcore_axis_name="s")


@functools.partial(
    pl.kernel,
    out_type=jax.ShapeDtypeStruct((N, NHID), jnp.float32),
    mesh=_mesh,
    scratch_types=[
        pltpu.VMEM((C,), jnp.int32),
        pltpu.VMEM((C, NHID), jnp.float32),
        pltpu.VMEM((C, NHID), jnp.float32),
        pltpu.SemaphoreType.DMA,
    ],
)
def _wl_pe(h_hbm, idx_hbm, w_hbm, out_hbm, idx_v, gat_v, acc_v, sem):
    wid = lax.axis_index("s") * NC + lax.axis_index("c")

    def do_chunk(c, rows):
        row0 = pl.multiple_of(c * C, C)
        if rows == C:
            idx_ref, gat_ref, acc_ref = idx_v, gat_v, acc_v
        else:
            idx_ref = idx_v.at[pl.ds(0, rows)]
            gat_ref = gat_v.at[pl.ds(0, rows)]
            acc_ref = acc_v.at[pl.ds(0, rows)]
        pltpu.sync_copy(idx_hbm.at[pl.ds(row0, rows)], idx_ref)
        pltpu.sync_copy(h_hbm.at[pl.ds(row0, rows)], acc_ref)
        pltpu.async_copy(w_hbm.at[idx_ref], gat_ref, sem).wait()

        def add_row(r, carry):
            for j in range(NHID // 16):
                sl = pl.ds(j * 16, 16)
                plsc.addupdate(acc_v.at[r, sl], gat_v[r, sl])
            return carry

        lax.fori_loop(0, rows, add_row, 0)
        pltpu.sync_copy(acc_ref, out_hbm.at[pl.ds(row0, rows)])

    def body(k, carry):
        do_chunk(wid + k * NW, C)
        return carry

    lax.fori_loop(0, CPW, body, 0)

    @pl.when(wid < EXTRA)
    def _extra():
        do_chunk(wid + CPW * NW, C)

    @pl.when(wid == TAILW)
    def _tail():
        do_chunk(jnp.int32(FULL), TAIL_ROWS)


def kernel(h, precomputed_eigenvectors, proj_weight):
    idx = precomputed_eigenvectors.astype(jnp.int32)
    return _wl_pe(h, idx, proj_weight)


# speedup vs baseline: 1.8548x; 1.8548x over previous
import functools

import jax
import jax.numpy as jnp
from jax import lax
from jax.experimental import pallas as pl
from jax.experimental.pallas import tpu as pltpu
from jax.experimental.pallas import tpu_sc as plsc

N = 100000
NHID = 128
NC = 2
NS = 16
NW = NC * NS
C = 128
FULL = N // C
CPW = FULL // NW
EXTRA = FULL - CPW * NW
TAIL_ROWS = N - FULL * C
KMAX = CPW + 1

_mesh = plsc.VectorSubcoreMesh(core_axis_name="c", subcore_axis_name="s")


@functools.partial(
    pl.kernel,
    out_type=jax.ShapeDtypeStruct((N, NHID), jnp.float32),
    mesh=_mesh,
    scratch_types=[
        pltpu.VMEM((KMAX, C), jnp.int32),
        pltpu.VMEM((C, NHID), jnp.float32),
        pltpu.VMEM((C, NHID), jnp.float32),
        pltpu.VMEM((C, NHID), jnp.float32),
        pltpu.VMEM((C, NHID), jnp.float32),
        pltpu.SemaphoreType.DMA,
        pltpu.SemaphoreType.DMA,
    ],
)
def _wl_pe(h_hbm, idx_hbm, w_hbm, out_hbm, idx_v, h0, h1, g0, g1, s0, s1):
    wid = lax.axis_index("s") * NC + lax.axis_index("c")
    pltpu.sync_copy(idx_hbm.at[wid], idx_v)

    hb, gb, sb = (h0, h1), (g0, g1), (s0, s1)

    def row0_of(k):
        return pl.multiple_of((k * NW + wid) * C, C)

    def start_in(k, b):
        r0 = row0_of(k)
        pltpu.async_copy(h_hbm.at[pl.ds(r0, C)], hb[b], sb[b])
        pltpu.async_copy(w_hbm.at[idx_v.at[k]], gb[b], sb[b])

    def wait_in(b):
        pltpu.make_async_copy(h_hbm.at[pl.ds(0, C)], hb[b], sb[b]).wait()
        pltpu.make_async_copy(h_hbm.at[pl.ds(0, C)], gb[b], sb[b]).wait()

    def add(b, rows=C):
        def add_row(r, carry):
            for j in range(NHID // 16):
                sl = pl.ds(j * 16, 16)
                plsc.addupdate(gb[b].at[r, sl], hb[b][r, sl])
            return carry

        lax.fori_loop(0, rows, add_row, 0)

    def store_out(k, b):
        pltpu.sync_copy(gb[b], out_hbm.at[pl.ds(row0_of(k), C)])

    start_in(0, 0)

    def pair(i, carry):
        k = 2 * i
        start_in(k + 1, 1)
        wait_in(0)
        add(0)
        store_out(k, 0)
        start_in(k + 2, 0)
        wait_in(1)
        add(1)
        store_out(k + 1, 1)
        return carry

    lax.fori_loop(0, CPW // 2 - 1, pair, 0)

    start_in(CPW - 1, 1)
    wait_in(0)
    add(0)
    store_out(CPW - 2, 0)

    @pl.when(wid < EXTRA)
    def _start_extra():
        start_in(CPW, 0)

    wait_in(1)
    add(1)
    store_out(CPW - 1, 1)

    @pl.when(wid < EXTRA)
    def _finish_extra():
        wait_in(0)
        add(0)
        store_out(CPW, 0)

    @pl.when(wid == EXTRA)
    def _tail():
        r0 = pl.multiple_of(FULL * C, C)
        t = pl.ds(0, TAIL_ROWS)
        pltpu.sync_copy(h_hbm.at[pl.ds(r0, TAIL_ROWS)], h0.at[t])
        pltpu.async_copy(w_hbm.at[idx_v.at[CPW, t]], g0.at[t], s0).wait()
        add(0, rows=TAIL_ROWS)
        pltpu.sync_copy(g0.at[t], out_hbm.at[pl.ds(r0, TAIL_ROWS)])


def kernel(h, precomputed_eigenvectors, proj_weight):
    idx = precomputed_eigenvectors.astype(jnp.int32)
    idxp = jnp.pad(idx, (0, KMAX * NW * C - N))
    idxp = idxp.reshape(KMAX, NW, C).transpose(1, 0, 2)
    return _wl_pe(h, idxp, proj_weight)
